# Initial kernel scaffold; baseline (speedup 1.0000x reference)
import jax
import jax.numpy as jnp
from jax.experimental import pallas as pl


def kernel(est_flow, gt_flow, gt_classes, gt_instance):
    raise NotImplementedError("write your pallas kernel here")


# speedup vs baseline: 2.7429x; 2.7429x over previous
import functools

import jax
import jax.numpy as jnp
from jax import lax
from jax.experimental import pallas as pl
from jax.experimental.pallas import tpu as pltpu
from jax.experimental.pallas import tpu_sc as plsc

_NC = 2
_NS = 16
_NW = _NC * _NS
_ACC = 1056
_ROWS = _ACC // 16


def _nrsqrt(x):
    i = plsc.bitcast(x, jnp.int32)
    i = jnp.int32(0x5F3759DF) - (i >> 1)
    y = plsc.bitcast(i, jnp.float32)
    h = x * jnp.float32(0.5)
    for _ in range(3):
        y = y * (jnp.float32(1.5) - (h * y) * y)
    return y


def _class_weight(m):
    f = jnp.float32
    return jnp.where(m == 0, f(0.1),
                     jnp.where(m == 1, f(1.0),
                               jnp.where(m == 2, f(2.0),
                                         jnp.where(m == 3, f(2.5), f(1.5)))))


def _pass1_body(chunk, last, est_ref, gt_ref, cls_ref, ins_ref, out_ref,
                est_v, gt_v, cls_v, ins_v, acc, red):
    wid = lax.axis_index("s") * _NC + lax.axis_index("c")
    iota = lax.iota(jnp.int32, 16)
    lane_off = iota * _ACC
    i3 = iota * 3
    is_last = wid == _NW - 1
    start = wid * chunk

    @pl.when(jnp.logical_not(is_last))
    def _():
        pltpu.sync_copy(est_ref.at[pl.ds(start * 3, chunk * 3)], est_v)
        pltpu.sync_copy(gt_ref.at[pl.ds(start * 3, chunk * 3)], gt_v)
        pltpu.sync_copy(cls_ref.at[pl.ds(start, chunk)], cls_v)
        pltpu.sync_copy(ins_ref.at[pl.ds(start, chunk)], ins_v)

    @pl.when(is_last)
    def _():
        pltpu.sync_copy(est_ref.at[pl.ds(start * 3, last * 3)],
                        est_v.at[pl.ds(0, last * 3)])
        pltpu.sync_copy(gt_ref.at[pl.ds(start * 3, last * 3)],
                        gt_v.at[pl.ds(0, last * 3)])
        pltpu.sync_copy(cls_ref.at[pl.ds(start, last)],
                        cls_v.at[pl.ds(0, last)])
        pltpu.sync_copy(ins_ref.at[pl.ds(start, last)],
                        ins_v.at[pl.ds(0, last)])

    zero16 = jnp.zeros((16,), jnp.float32)
    ones16 = jnp.ones((16,), jnp.float32)

    def zbody(j, carry):
        base = j * 64
        for t in range(4):
            acc[pl.ds(base + t * 16, 16)] = zero16
        return carry

    lax.fori_loop(0, (16 * _ACC) // 64, zbody, 0)

    n_groups = jnp.where(is_last, last // 16, chunk // 16)
    fmax = jnp.float32(3.4028235e38)

    def gbody(g, carry):
        g48 = g * 48
        ex = plsc.load_gather(est_v, [g48 + i3])
        ey = plsc.load_gather(est_v, [g48 + i3 + 1])
        ez = plsc.load_gather(est_v, [g48 + i3 + 2])
        gx = plsc.load_gather(gt_v, [g48 + i3])
        gy = plsc.load_gather(gt_v, [g48 + i3 + 1])
        gz = plsc.load_gather(gt_v, [g48 + i3 + 2])
        cls = cls_v[pl.ds(g * 16, 16)]
        ins = ins_v[pl.ds(g * 16, 16)]
        dx = ex - gx
        dy = ey - gy
        dz = ez - gz
        ss = dx * dx + dy * dy + dz * dz
        gs = gx * gx + gy * gy + gz * gz
        pts = ss * _nrsqrt(ss)
        spd = (gs * _nrsqrt(gs)) * jnp.float32(10.0)
        fin = ((jnp.abs(ex) <= fmax) & (jnp.abs(ey) <= fmax)
               & (jnp.abs(ez) <= fmax) & (jnp.abs(gx) <= fmax)
               & (jnp.abs(gy) <= fmax) & (jnp.abs(gz) <= fmax))
        fmf = jnp.where(fin, jnp.float32(1.0), jnp.float32(0.0))
        pw = jnp.where(fin, pts, jnp.float32(0.0))
        sw = jnp.where(fin, spd, jnp.float32(0.0))
        b = ((spd >= jnp.float32(0.4)).astype(jnp.int32)
             + (spd > jnp.float32(1.0)).astype(jnp.int32))
        veh = ((cls >= 7) & (cls <= 10)) | (cls == 12) | (cls == 13)
        ped = (cls >= 2) & (cls <= 4)
        whl = (cls == 6) | (cls == 11)
        meta = jnp.where(cls == 0, 0,
                         jnp.where(veh, 1,
                                   jnp.where(ped, 2,
                                             jnp.where(whl, 3, 4))))
        i1 = lane_off + ins
        plsc.addupdate_scatter(acc, [i1], fmf)
        plsc.addupdate_scatter(acc, [i1 + 128], pw)
        plsc.addupdate_scatter(acc, [i1 + 256], sw)
        plsc.addupdate_scatter(acc, [i1 + 384 + meta * 128], ones16)
        i5 = lane_off + 1024 + meta * 3 + b
        plsc.addupdate_scatter(acc, [i5], pw)
        plsc.addupdate_scatter(acc, [i5 + 16], fmf)
        return carry

    lax.fori_loop(0, n_groups, gbody, 0)

    def rbody(e, carry):
        off = e * 16
        t = acc[pl.ds(off, 16)]
        for l in range(1, 16):
            t = t + acc[pl.ds(l * _ACC + off, 16)]
        red[pl.ds(off, 16)] = t
        return carry

    lax.fori_loop(0, _ROWS, rbody, 0)
    pltpu.sync_copy(red, out_ref.at[wid])


def _pass2_body(parts_ref, out_ref, parts_v, tbl_v, out_v):
    wid = lax.axis_index("s") * _NC + lax.axis_index("c")

    @pl.when(wid == 0)
    def _():
        pltpu.sync_copy(parts_ref, parts_v)
        iota = lax.iota(jnp.int32, 16)
        f = jnp.float32

        def rbody(e, carry):
            off = e * 16
            t = parts_v[pl.ds(off, 16)]
            for w in range(1, _NW):
                t = t + parts_v[pl.ds(w * _ACC + off, 16)]
            tbl_v[pl.ds(off, 16)] = t
            return carry

        lax.fori_loop(0, _ROWS, rbody, 0)

        terms = jnp.zeros((16,), jnp.float32)
        act = jnp.zeros((16,), jnp.float32)
        for q in range(8):
            off = q * 16
            cnt = tbl_v[pl.ds(off, 16)]
            spts = tbl_v[pl.ds(128 + off, 16)]
            sspd = tbl_v[pl.ds(256 + off, 16)]
            safe = jnp.maximum(cnt, f(1.0))
            err = jnp.where(cnt > 0, spts / safe, f(0.0))
            smean = jnp.where(cnt > 0, sspd / safe, f(0.0))
            best = tbl_v[pl.ds(384 + off, 16)]
            mode = jnp.zeros((16,), jnp.int32)
            for k in range(1, 5):
                ck = tbl_v[pl.ds(384 + k * 128 + off, 16)]
                better = ck > best
                best = jnp.where(better, ck, best)
                mode = jnp.where(better, k, mode)
            wv = _class_weight(mode)
            inst_id = off + iota
            gate = (inst_id > 0) & (cnt > 0) & (smean > f(0.4))
            gf = jnp.where(gate, f(1.0), f(0.0))
            terms = terms + gf * err * jnp.exp(err) * wv
            act = act + gf
        ones16 = jnp.ones((16,), jnp.float32)
        n_act_v = jnp.sum(act) * ones16
        t_sum_v = jnp.sum(terms) * ones16
        inst_loss_v = jnp.where(n_act_v > 0,
                                t_sum_v / jnp.maximum(n_act_v, f(1.0)),
                                f(0.0))

        s_tbl = tbl_v[pl.ds(1024, 16)]
        c_tbl = tbl_v[pl.ds(1040, 16)]
        valid = iota < 15
        b_id = iota % 3
        m_id = iota // 3
        mean_cb = jnp.where(c_tbl > 0, s_tbl / jnp.maximum(c_tbl, f(1.0)),
                            f(0.0))
        bcoef = jnp.where(b_id == 0, f(0.1), jnp.where(b_id == 1, f(0.4),
                                                       f(0.5)))
        wcls = _class_weight(m_id)
        class_loss_v = (jnp.sum(jnp.where(valid, mean_cb * bcoef * wcls,
                                          f(0.0))) * ones16)
        base_v = jnp.zeros((16,), jnp.float32)
        for b in range(3):
            msk = valid & (b_id == b)
            sb_v = jnp.sum(jnp.where(msk, s_tbl, f(0.0))) * ones16
            cb_v = jnp.sum(jnp.where(msk, c_tbl, f(0.0))) * ones16
            base_v = base_v + jnp.where(cb_v > 0,
                                        sb_v / jnp.maximum(cb_v, f(1.0)),
                                        f(0.0))
        total_v = base_v + class_loss_v + inst_loss_v
        out_v[...] = jnp.where(iota == 0, total_v, f(0.0))
        pltpu.sync_copy(out_v, out_ref)


@functools.lru_cache(maxsize=None)
def _build(n):
    assert n % 16 == 0, "point count must be a multiple of 16"
    chunk = -(-n // _NW)
    chunk = (chunk + 15) // 16 * 16
    last = n - (_NW - 1) * chunk
    assert 0 < last <= chunk and last % 16 == 0

    mesh = plsc.VectorSubcoreMesh(core_axis_name="c", subcore_axis_name="s")
    params = pltpu.CompilerParams(needs_layout_passes=False)
    pass1 = pl.kernel(
        functools.partial(_pass1_body, chunk, last),
        out_type=jax.ShapeDtypeStruct((_NW, _ACC), jnp.float32),
        mesh=mesh,
        compiler_params=params,
        scratch_types=[
            pltpu.VMEM((chunk * 3,), jnp.float32),
            pltpu.VMEM((chunk * 3,), jnp.float32),
            pltpu.VMEM((chunk,), jnp.int32),
            pltpu.VMEM((chunk,), jnp.int32),
            pltpu.VMEM((16 * _ACC,), jnp.float32),
            pltpu.VMEM((_ACC,), jnp.float32),
        ],
    )
    pass2 = pl.kernel(
        _pass2_body,
        out_type=jax.ShapeDtypeStruct((16,), jnp.float32),
        mesh=mesh,
        compiler_params=params,
        scratch_types=[
            pltpu.VMEM((_NW * _ACC,), jnp.float32),
            pltpu.VMEM((_ACC,), jnp.float32),
            pltpu.VMEM((16,), jnp.float32),
        ],
    )
    return pass1, pass2


def kernel(est_flow, gt_flow, gt_classes, gt_instance):
    n = est_flow.shape[0]
    pass1, pass2 = _build(n)
    parts = pass1(est_flow.reshape(-1), gt_flow.reshape(-1),
                  gt_classes.astype(jnp.int32), gt_instance.astype(jnp.int32))
    out = pass2(parts.reshape(-1))
    return out[0]
